# baseline (speedup 1.0000x reference)
import numpy as np
import jax
import jax.numpy as jnp
from jax.experimental import pallas as pl
from jax.experimental.pallas import tpu as pltpu

ALPHA = 4
FB = 8


def _pack_body(in_ref, slow_hbm, fast_ref, sem0, sem1, *, a, b):
    s = pl.program_id(0)
    loc0 = (2 * s * a) // b - FB * s
    loc1 = ((2 * s + 1) * a) // b - FB * s
    cp0 = pltpu.make_async_copy(
        in_ref.at[:, pl.ds(loc0, 1)], slow_hbm.at[:, pl.ds(2 * s, 1)], sem0
    )
    cp1 = pltpu.make_async_copy(
        in_ref.at[:, pl.ds(loc1, 1)], slow_hbm.at[:, pl.ds(2 * s + 1, 1)], sem1
    )
    cp0.start()
    cp1.start()
    fast_ref[...] = in_ref[...]
    cp0.wait()
    cp1.wait()


def kernel(frames):
    C, T, H, W = frames.shape
    N = T // ALPHA
    a, b = T - 1, N - 1

    idx = np.linspace(0, T - 1, N).astype(np.int64)
    idx_arith = (np.arange(N) * a) // b
    assert np.array_equal(idx, idx_arith), (idx, idx_arith)
    assert np.array_equal(idx // FB, np.arange(N) // 2), idx

    slow, fast = pl.pallas_call(
        lambda i, so, fo, s0, s1: _pack_body(i, so, fo, s0, s1, a=a, b=b),
        grid=(T // FB,),
        in_specs=[pl.BlockSpec((C, FB, H, W), lambda s: (0, s, 0, 0))],
        out_specs=(
            pl.BlockSpec(memory_space=pltpu.MemorySpace.HBM),
            pl.BlockSpec((C, FB, H, W), lambda s: (0, s, 0, 0)),
        ),
        out_shape=(
            jax.ShapeDtypeStruct((C, N, H, W), frames.dtype),
            jax.ShapeDtypeStruct((C, T, H, W), frames.dtype),
        ),
        scratch_shapes=[pltpu.SemaphoreType.DMA, pltpu.SemaphoreType.DMA],
    )(frames)
    return (slow, fast)


# speedup vs baseline: 1.0466x; 1.0466x over previous
import numpy as np
import jax
import jax.numpy as jnp
from jax.experimental import pallas as pl
from jax.experimental.pallas import tpu as pltpu

ALPHA = 4
FB = 8


def _pack_body(in_ref, slow_ref, fast_ref, *, a, b):
    s = pl.program_id(0)
    fast_ref[...] = in_ref[...]
    loc0 = (2 * s * a) // b - FB * s
    loc1 = ((2 * s + 1) * a) // b - FB * s
    slow_ref[:, pl.ds(0, 1)] = in_ref[:, pl.ds(loc0, 1)]
    slow_ref[:, pl.ds(1, 1)] = in_ref[:, pl.ds(loc1, 1)]


def kernel(frames):
    C, T, H, W = frames.shape
    N = T // ALPHA
    a, b = T - 1, N - 1

    idx = np.linspace(0, T - 1, N).astype(np.int64)
    idx_arith = (np.arange(N) * a) // b
    assert np.array_equal(idx, idx_arith), (idx, idx_arith)
    assert np.array_equal(idx // FB, np.arange(N) // 2), idx

    slow, fast = pl.pallas_call(
        lambda i, s, f: _pack_body(i, s, f, a=a, b=b),
        grid=(T // FB,),
        in_specs=[pl.BlockSpec((C, FB, H, W), lambda s: (0, s, 0, 0))],
        out_specs=(
            pl.BlockSpec((C, 2, H, W), lambda s: (0, s, 0, 0)),
            pl.BlockSpec((C, FB, H, W), lambda s: (0, s, 0, 0)),
        ),
        out_shape=(
            jax.ShapeDtypeStruct((C, N, H, W), frames.dtype),
            jax.ShapeDtypeStruct((C, T, H, W), frames.dtype),
        ),
        compiler_params=pltpu.CompilerParams(vmem_limit_bytes=100 * 2**20),
    )(frames)
    return (slow, fast)
